# baseline (speedup 1.0000x reference)
import functools

import jax
import jax.numpy as jnp
from jax import lax
from jax.experimental import pallas as pl
from jax.experimental.pallas import tpu as pltpu
from jax.experimental.pallas import tpu_sc as plsc

L = 256
D = 32
NBUF = 4
M = 2


def _make_gather(n_rows: int, n_chunks: int):
    info = plsc.get_sparse_core_info()
    nw = info.num_cores * info.num_subcores
    rows_per_w = n_chunks // nw
    assert rows_per_w % NBUF == 0 and rows_per_w >= 3 * NBUF
    mesh = plsc.VectorSubcoreMesh(core_axis_name="c", subcore_axis_name="s")

    @functools.partial(
        pl.kernel,
        out_type=jax.ShapeDtypeStruct((n_rows, D), jnp.float32),
        mesh=mesh,
        scratch_types=[
            pltpu.VMEM((rows_per_w, L), jnp.int32),
            pltpu.VMEM((NBUF, L, D), jnp.float32),
        ]
        + [pltpu.SemaphoreType.DMA] * (2 * NBUF),
        compiler_params=pltpu.CompilerParams(use_tc_tiling_on_sc=False),
    )
    def k(idx_hbm, table_hbm, out_hbm, idx_v, rows_v, *sems):
        gsem = sems[:NBUF]
        ssem = sems[NBUF:]
        wid = lax.axis_index("s") * info.num_cores + lax.axis_index("c")
        base = wid * rows_per_w
        pltpu.sync_copy(idx_hbm.at[pl.ds(base, rows_per_w)], idx_v)

        def g_start(jj, b):
            pltpu.async_copy(table_hbm.at[idx_v.at[jj]], rows_v.at[b], gsem[b])

        def g_wait(jj, b):
            pltpu.make_async_copy(
                table_hbm.at[idx_v.at[jj]], rows_v.at[b], gsem[b]
            ).wait()

        def s_start(jj, b):
            pltpu.async_copy(
                rows_v.at[b], out_hbm.at[pl.ds((base + jj) * L, L)], ssem[b]
            )

        def s_wait(jj, b):
            pltpu.make_async_copy(
                rows_v.at[b], out_hbm.at[pl.ds((base + jj) * L, L)], ssem[b]
            ).wait()

        def step(jj, b, issue_next, wait_prev_store):
            g_wait(jj, b)
            s_start(jj, b)
            if issue_next:
                b2 = (b + M) % NBUF
                if wait_prev_store:
                    s_wait(jj + M - NBUF, b2)
                g_start(jj + M, b2)

        for kk in range(M):
            g_start(kk, kk)

        for b in range(NBUF):
            step(b, b, issue_next=True, wait_prev_store=(b + M >= NBUF))

        @pl.loop(NBUF, rows_per_w - NBUF, step=NBUF)
        def _(j):
            for b in range(NBUF):
                step(j + b, b, issue_next=True, wait_prev_store=True)

        for b in range(NBUF):
            jj = rows_per_w - NBUF + b
            step(jj, b, issue_next=(b < NBUF - M), wait_prev_store=True)

        for b in range(NBUF):
            s_wait(rows_per_w - NBUF + b, b)

    return k


def kernel(idx, table):
    b0, b1 = idx.shape
    n = b0 * b1
    idx2 = idx.reshape(n // L, L).astype(jnp.int32)
    out = _make_gather(n, n // L)(idx2, table)
    return out.reshape(b0, b1, D)


# speedup vs baseline: 1.3395x; 1.3395x over previous
import functools

import jax
import jax.numpy as jnp
from jax import lax
from jax.experimental import pallas as pl
from jax.experimental.pallas import tpu as pltpu
from jax.experimental.pallas import tpu_sc as plsc

LANE = 128
D = 32
NBUF = 4
M = 2


def _make_gather(n_vocab: int, b0: int, b1: int):
    info = plsc.get_sparse_core_info()
    nw = info.num_cores * info.num_subcores
    bw = b0 // nw
    nq = bw // LANE
    assert nq == NBUF and b0 % (LANE * nw) == 0 and D % 8 == 0
    ntc = b0 // LANE
    mesh = plsc.VectorSubcoreMesh(core_axis_name="c", subcore_axis_name="s")

    @functools.partial(
        pl.kernel,
        out_type=jax.ShapeDtypeStruct((b1, D // 8, ntc, 8, LANE), jnp.float32),
        mesh=mesh,
        scratch_types=[
            pltpu.VMEM((b1, bw), jnp.int32),
            pltpu.VMEM((NBUF, LANE, D), jnp.float32),
            pltpu.VMEM((NBUF, D // 8, 1, 8, LANE), jnp.float32),
        ]
        + [pltpu.SemaphoreType.DMA] * (2 * NBUF),
        compiler_params=pltpu.CompilerParams(
            use_tc_tiling_on_sc=False, needs_layout_passes=False
        ),
    )
    def k(idxt_hbm, table_hbm, out_hbm, idx_v, rows_v, t_v, *sems):
        gsem = sems[:NBUF]
        ssem = sems[NBUF:]
        wid = lax.axis_index("s") * info.num_cores + lax.axis_index("c")
        pltpu.sync_copy(idxt_hbm.at[:, pl.ds(wid * bw, bw)], idx_v)

        iotas = [lax.iota(jnp.int32, 16) + 16 * kk for kk in range(LANE // 16)]
        cols = [jnp.full((16,), d, jnp.int32) for d in range(D)]

        def g_start(j, q):
            pltpu.async_copy(
                table_hbm.at[idx_v.at[j, pl.ds(q * LANE, LANE)]],
                rows_v.at[q], gsem[q],
            )

        def g_wait(j, q):
            pltpu.make_async_copy(
                table_hbm.at[idx_v.at[j, pl.ds(q * LANE, LANE)]],
                rows_v.at[q], gsem[q],
            ).wait()

        def s_start(j, q):
            pltpu.async_copy(
                t_v.at[q], out_hbm.at[j, :, pl.ds(nq * wid + q, 1)], ssem[q],
            )

        def s_wait(j, q):
            pltpu.make_async_copy(
                t_v.at[q], out_hbm.at[j, :, pl.ds(nq * wid + q, 1)], ssem[q],
            ).wait()

        def transpose(q):
            for d in range(D):
                for kk in range(LANE // 16):
                    v = plsc.load_gather(rows_v.at[q], [iotas[kk], cols[d]])
                    t_v[q, d // 8, 0, d % 8, pl.ds(16 * kk, 16)] = v

        def step(j, q, wait_store, prefetch):
            g_wait(j, q)
            if wait_store:
                s_wait(j - 1, q)
            transpose(q)
            s_start(j, q)
            if prefetch:
                j2, q2 = (j, q + M) if q + M < nq else (j + 1, q + M - nq)
                g_start(j2, q2)

        for q in range(M):
            g_start(0, q)
        for q in range(nq):
            step(0, q, wait_store=False, prefetch=True)

        @pl.loop(1, b1 - 1)
        def _(j):
            for q in range(nq):
                step(j, q, wait_store=True, prefetch=True)

        for q in range(nq):
            step(b1 - 1, q, wait_store=True, prefetch=(q + M < nq))
        for q in range(nq):
            s_wait(b1 - 1, q)

    return k


def kernel(idx, table):
    b0, b1 = idx.shape
    n_vocab, d = table.shape
    idxt = jnp.transpose(idx.astype(jnp.int32), (1, 0))
    w = _make_gather(n_vocab, b0, b1)(idxt, table)
    return jnp.transpose(w, (2, 4, 0, 1, 3)).reshape(b0, b1, d)


# speedup vs baseline: 2.0916x; 1.5614x over previous
import functools

import jax
import jax.numpy as jnp
from jax import lax
from jax.experimental import pallas as pl
from jax.experimental.pallas import tpu as pltpu
from jax.experimental.pallas import tpu_sc as plsc

LANE = 128
D = 32
CP = 129
NBUF = 4
M = 3


def _make_gather(n_vocab: int, b0: int, b1: int):
    info = plsc.get_sparse_core_info()
    nw = info.num_cores * info.num_subcores
    bw = b0 // nw
    nq = bw // LANE
    assert nq == NBUF and b0 % (LANE * nw) == 0 and D % 8 == 0
    ntc = b0 // LANE
    ntr = (b1 + 7) // 8
    mesh = plsc.VectorSubcoreMesh(core_axis_name="c", subcore_axis_name="s")

    @functools.partial(
        pl.kernel,
        out_type=jax.ShapeDtypeStruct((b1, D // 8, ntc, 8, LANE), jnp.float32),
        mesh=mesh,
        scratch_types=[
            pltpu.VMEM((ntr, nq, 8, LANE), jnp.int32),
            pltpu.VMEM((NBUF, LANE, D), jnp.float32),
            pltpu.VMEM((NBUF, D // 8, 1, 8, CP), jnp.float32),
        ]
        + [pltpu.SemaphoreType.DMA] * (2 * NBUF),
        compiler_params=pltpu.CompilerParams(
            use_tc_tiling_on_sc=False, needs_layout_passes=False
        ),
    )
    def k(idxt_hbm, table_hbm, out_hbm, idx_v, rows_v, t_v, *sems):
        gsem = sems[:NBUF]
        ssem = sems[NBUF:]
        wid = lax.axis_index("s") * info.num_cores + lax.axis_index("c")
        pltpu.sync_copy(idxt_hbm.at[:, pl.ds(nq * wid, nq), :, :], idx_v)

        dvec = [lax.iota(jnp.int32, 16) + 16 * g for g in range(D // 16)]
        trv = [v // 8 for v in dvec]
        rv = [v % 8 for v in dvec]
        zv = jnp.full((16,), 0, jnp.int32)

        def ilist(j, q):
            return idx_v.at[j // 8, q, j % 8, :]

        def g_start(j, q):
            pltpu.async_copy(
                table_hbm.at[ilist(j, q)], rows_v.at[q], gsem[q],
            )

        def g_wait(j, q):
            pltpu.make_async_copy(
                table_hbm.at[ilist(j, q)], rows_v.at[q], gsem[q],
            ).wait()

        def s_start(j, q):
            pltpu.async_copy(
                t_v.at[q, :, :, :, pl.ds(0, LANE)],
                out_hbm.at[j, :, pl.ds(nq * wid + q, 1)], ssem[q],
            )

        def s_wait(j, q):
            pltpu.make_async_copy(
                t_v.at[q, :, :, :, pl.ds(0, LANE)],
                out_hbm.at[j, :, pl.ds(nq * wid + q, 1)], ssem[q],
            ).wait()

        def transpose(q):
            for c in range(LANE):
                cv = jnp.full((16,), c, jnp.int32)
                for g in range(D // 16):
                    v = rows_v[q, c, pl.ds(16 * g, 16)]
                    plsc.store_scatter(
                        t_v.at[q], [trv[g], zv, rv[g], cv], v
                    )

        def step(j, q, wait_store, prefetch):
            g_wait(j, q)
            if wait_store:
                s_wait(j - 1, q)
            transpose(q)
            s_start(j, q)
            if prefetch:
                q2 = (q + M) % nq
                g_start(j + (q + M) // nq, q2)

        for q in range(M):
            g_start(0, q)
        for q in range(nq):
            step(0, q, wait_store=False, prefetch=True)

        @pl.loop(1, b1 - 1)
        def _(j):
            for q in range(nq):
                step(j, q, wait_store=True, prefetch=True)

        for q in range(nq):
            step(b1 - 1, q, wait_store=True, prefetch=(q + M < nq))
        for q in range(nq):
            s_wait(b1 - 1, q)

    return k


def kernel(idx, table):
    b0, b1 = idx.shape
    n_vocab, d = table.shape
    ntr = (b1 + 7) // 8
    idxt = jnp.transpose(idx.astype(jnp.int32), (1, 0))
    idxp = jnp.pad(idxt, ((0, 8 * ntr - b1), (0, 0)))
    idx4 = jnp.transpose(
        idxp.reshape(ntr, 8, b0 // LANE, LANE), (0, 2, 1, 3)
    )
    w = _make_gather(n_vocab, b0, b1)(idx4, table)
    return jnp.transpose(w, (2, 4, 0, 1, 3)).reshape(b0, b1, d)


# speedup vs baseline: 2.0926x; 1.0005x over previous
import functools

import jax
import jax.numpy as jnp
from jax import lax
from jax.experimental import pallas as pl
from jax.experimental.pallas import tpu as pltpu
from jax.experimental.pallas import tpu_sc as plsc

LANE = 128
D = 32
CP = 129
NBUF = 4
M = 3


def _make_gather(n_vocab: int, b0: int, b1: int):
    info = plsc.get_sparse_core_info()
    nw = info.num_cores * info.num_subcores
    bw = b0 // nw
    nq = bw // LANE
    assert nq == NBUF and b0 % (LANE * nw) == 0 and D % 8 == 0
    ntc = b0 // LANE
    ntr = (b1 + 7) // 8
    mesh = plsc.VectorSubcoreMesh(core_axis_name="c", subcore_axis_name="s")

    @functools.partial(
        pl.kernel,
        out_type=jax.ShapeDtypeStruct((b1, D // 8, ntc, 8, LANE), jnp.float32),
        mesh=mesh,
        scratch_types=[
            pltpu.VMEM((ntr, nq, 8, LANE), jnp.int32),
            pltpu.VMEM((NBUF, LANE, D), jnp.float32),
            pltpu.VMEM((NBUF, D // 8, 1, 8, CP), jnp.float32),
        ]
        + [pltpu.SemaphoreType.DMA] * (2 * NBUF),
        compiler_params=pltpu.CompilerParams(
            use_tc_tiling_on_sc=False, needs_layout_passes=False
        ),
    )
    def k(idxt_hbm, table_hbm, out_hbm, idx_v, rows_v, t_v, *sems):
        gsem = sems[:NBUF]
        ssem = sems[NBUF:]
        wid = lax.axis_index("s") * info.num_cores + lax.axis_index("c")
        pltpu.sync_copy(idxt_hbm.at[:, pl.ds(nq * wid, nq), :, :], idx_v)

        dvec = [lax.iota(jnp.int32, 16) + 16 * g for g in range(D // 16)]
        trv = [v // 8 for v in dvec]
        rv = [v % 8 for v in dvec]
        zv = jnp.full((16,), 0, jnp.int32)

        def ilist(j, q):
            return idx_v.at[j // 8, q, j % 8, :]

        def g_start(j, q):
            pltpu.async_copy(
                table_hbm.at[ilist(j, q)], rows_v.at[q], gsem[q],
            )

        def g_wait(j, q):
            pltpu.make_async_copy(
                table_hbm.at[ilist(j, q)], rows_v.at[q], gsem[q],
            ).wait()

        def s_start(j, q):
            pltpu.async_copy(
                t_v.at[q, :, :, :, pl.ds(0, LANE)],
                out_hbm.at[j, :, pl.ds(nq * wid + q, 1)], ssem[q],
            )

        def s_wait(j, q):
            pltpu.make_async_copy(
                t_v.at[q, :, :, :, pl.ds(0, LANE)],
                out_hbm.at[j, :, pl.ds(nq * wid + q, 1)], ssem[q],
            ).wait()

        def transpose(q):
            for c in range(LANE):
                cv = jnp.full((16,), c, jnp.int32)
                for g in range(D // 16):
                    v = rows_v[q, c, pl.ds(16 * g, 16)]
                    plsc.store_scatter(
                        t_v.at[q], [trv[g], zv, rv[g], cv], v
                    )

        def step(j, q, wait_store, prefetch):
            g_wait(j, q)
            if wait_store:
                s_wait(j - 1, q)
            transpose(q)
            s_start(j, q)
            if prefetch:
                q2 = (q + M) % nq
                g_start(j + (q + M) // nq, q2)

        for q in range(M):
            g_start(0, q)
        for q in range(nq):
            step(0, q, wait_store=False, prefetch=True)

        @pl.loop(1, b1 - 1)
        def _(j):
            for q in range(nq):
                step(j, q, wait_store=True, prefetch=True)

        for q in range(nq):
            step(b1 - 1, q, wait_store=True, prefetch=(q + M < nq))
        for q in range(nq):
            s_wait(b1 - 1, q)

    return k


def kernel(idx, table):
    b0, b1 = idx.shape
    n_vocab, d = table.shape
    ntr = (b1 + 7) // 8
    idxt = jnp.transpose(idx.astype(jnp.int32), (1, 0))
    idxp = jnp.pad(idxt, ((0, 8 * ntr - b1), (0, 0)))
    idxp = lax.optimization_barrier(idxp)
    idx4 = jnp.transpose(
        idxp.reshape(ntr, 8, b0 // LANE, LANE), (0, 2, 1, 3)
    )
    w = _make_gather(n_vocab, b0, b1)(idx4, table)
    return jnp.transpose(w, (2, 4, 0, 1, 3)).reshape(b0, b1, d)
